# baseline (speedup 1.0000x reference)
import functools

import jax
import jax.numpy as jnp
from jax import lax
from jax.experimental import pallas as pl
from jax.experimental.pallas import tpu as pltpu
from jax.experimental.pallas import tpu_sc as plsc

NN = 10000
EE = 320000
DD = 128
NC = 2
NS = 16
NW = NC * NS
EPW = EE // NW
CH = 40
NCH = EPW // CH
SLAB = 624
SLAB_LAST = NN - SLAB * (NS - 1)
NBUF = 6
NFULL = (NCH // NBUF) * NBUF
MROWS = 1000


def _sc_spmm(table, src3, dst3, zer):
    mesh = plsc.VectorSubcoreMesh(core_axis_name="c", subcore_axis_name="s")

    @functools.partial(
        pl.kernel,
        out_type=[
            jax.ShapeDtypeStruct((NN, DD), jnp.float32),
            jax.ShapeDtypeStruct((NN, DD), jnp.float32),
        ],
        mesh=mesh,
        scratch_types=[
            pltpu.VMEM((NCH, CH), jnp.int32),
            pltpu.VMEM((NCH, CH), jnp.int32),
            [pltpu.VMEM((CH, DD), jnp.float32) for _ in range(NBUF)],
            [pltpu.SemaphoreType.DMA for _ in range(NBUF)],
            [pltpu.SemaphoreType.DMA for _ in range(NBUF)],
            pltpu.VMEM_SHARED((NN, DD), jnp.float32),
            pltpu.SemaphoreType.DMA,
        ],
        compiler_params=pltpu.CompilerParams(use_tc_tiling_on_sc=False),
    )
    def spmm(table_hbm, src_hbm, dst_hbm, zer_hbm, out0, out1,
             src_v, dst_v, bufs, semg, sems, acc, semi):
        cid = lax.axis_index("c")
        sid = lax.axis_index("s")
        wid = cid * NS + sid
        base = pl.multiple_of(sid * SLAB, 8)

        cpi0 = pltpu.async_copy(src_hbm.at[wid], src_v, semi)
        cpi1 = pltpu.async_copy(dst_hbm.at[wid], dst_v, semi)

        zb = bufs[NBUF - 1]
        nz = SLAB // CH
        rem = SLAB - nz * CH
        pltpu.sync_copy(zer_hbm, zb)
        cpi0.wait()
        for b in range(NBUF - 1):
            pltpu.async_copy(table_hbm.at[src_v.at[b]], bufs[b], semg[b])

        for k in range(nz):
            off = pl.multiple_of(base + k * CH, 8)
            pltpu.async_copy(zb, acc.at[pl.ds(off, CH)], sems[k % NBUF])
        offr = pl.multiple_of(base + nz * CH, 8)
        pltpu.async_copy(zb.at[pl.ds(0, rem)], acc.at[pl.ds(offr, rem)],
                         sems[nz % NBUF])

        @pl.when(sid == NS - 1)
        def _():
            off = pl.multiple_of(base + SLAB, 8)
            pltpu.sync_copy(zb.at[pl.ds(0, SLAB_LAST - SLAB)],
                            acc.at[pl.ds(off, SLAB_LAST - SLAB)])

        for k in range(nz):
            off = pl.multiple_of(base + k * CH, 8)
            pltpu.make_async_copy(zb, acc.at[pl.ds(off, CH)],
                                  sems[k % NBUF]).wait()
        pltpu.make_async_copy(zb.at[pl.ds(0, rem)],
                              acc.at[pl.ds(offr, rem)],
                              sems[nz % NBUF]).wait()
        cpi1.wait()
        plsc.subcore_barrier()

        pltpu.async_copy(table_hbm.at[src_v.at[NBUF - 1]],
                         bufs[NBUF - 1], semg[NBUF - 1])

        def body(g, carry):
            c0 = g * NBUF
            for b in range(NBUF):
                c = c0 + b
                pltpu.make_async_copy(table_hbm.at[src_v.at[c]],
                                      bufs[b], semg[b]).wait()
                pltpu.async_copy(bufs[b], acc.at[dst_v.at[c]], sems[b],
                                 add=True)
            for b in range(NBUF):
                nc = c0 + b + NBUF

                @pl.when(nc < NCH)
                def _():
                    pltpu.make_async_copy(bufs[b], acc.at[dst_v.at[c0 + b]],
                                          sems[b]).wait()
                    pltpu.async_copy(table_hbm.at[src_v.at[nc]],
                                     bufs[b], semg[b])
            return carry

        lax.fori_loop(0, NCH // NBUF, body, 0)
        for c in range(NFULL, NCH):
            b = c % NBUF
            pltpu.make_async_copy(table_hbm.at[src_v.at[c]],
                                  bufs[b], semg[b]).wait()
            pltpu.async_copy(bufs[b], acc.at[dst_v.at[c]], sems[b], add=True)
        for c in range(NCH - NBUF, NCH):
            b = c % NBUF
            pltpu.make_async_copy(bufs[b], acc.at[dst_v.at[c]],
                                  sems[b]).wait()
        plsc.subcore_barrier()

        def writeback(out):
            @pl.when(sid < NS - 1)
            def _():
                pltpu.sync_copy(acc.at[pl.ds(base, SLAB)],
                                out.at[pl.ds(base, SLAB)])

            @pl.when(sid == NS - 1)
            def _():
                pltpu.sync_copy(acc.at[pl.ds(base, SLAB_LAST)],
                                out.at[pl.ds(base, SLAB_LAST)])

        @pl.when(cid == 0)
        def _():
            writeback(out0)

        @pl.when(cid == 1)
        def _():
            writeback(out1)

    return spmm(table, src3, dst3, zer)


def _combine_mm(p0, p1, W, b, relu):
    def body(p0_ref, p1_ref, w_ref, b_ref, o_ref):
        s = jnp.dot(p0_ref[...] + p1_ref[...], w_ref[...],
                    preferred_element_type=jnp.float32,
                    precision=jax.lax.Precision.HIGHEST) + b_ref[...]
        if relu:
            s = jnp.maximum(s, 0.0)
        o_ref[...] = s

    return pl.pallas_call(
        body,
        grid=(NN // MROWS,),
        in_specs=[
            pl.BlockSpec((MROWS, DD), lambda i: (i, 0)),
            pl.BlockSpec((MROWS, DD), lambda i: (i, 0)),
            pl.BlockSpec((DD, DD), lambda i: (0, 0)),
            pl.BlockSpec((1, DD), lambda i: (0, 0)),
        ],
        out_specs=pl.BlockSpec((MROWS, DD), lambda i: (i, 0)),
        out_shape=jax.ShapeDtypeStruct((NN, DD), jnp.float32),
    )(p0, p1, W, b)


def kernel(x, edge_index, W1, b1, W2, b2):
    src3 = edge_index[0].reshape(NW, NCH, CH)
    dst3 = edge_index[1].reshape(NW, NCH, CH)
    zer = jnp.zeros((CH, DD), jnp.float32)
    p0, p1 = _sc_spmm(x, src3, dst3, zer)
    h = _combine_mm(p0, p1, W1, b1.reshape(1, DD), relu=True)
    q0, q1 = _sc_spmm(h, src3, dst3, zer)
    return _combine_mm(q0, q1, W2, b2.reshape(1, DD), relu=False)


# speedup vs baseline: 1.0057x; 1.0057x over previous
import functools

import jax
import jax.numpy as jnp
from jax import lax
from jax.experimental import pallas as pl
from jax.experimental.pallas import tpu as pltpu
from jax.experimental.pallas import tpu_sc as plsc

NN = 10000
EE = 320000
DD = 128
NC = 2
NS = 16
NW = NC * NS
EPW = EE // NW
CH = 40
NCH = EPW // CH
SLAB = 624
SLAB_LAST = NN - SLAB * (NS - 1)
NBUF = 5
NFULL = (NCH // NBUF) * NBUF
MROWS = 1000


def _sc_spmm(table, src3, dst3, zer):
    mesh = plsc.VectorSubcoreMesh(core_axis_name="c", subcore_axis_name="s")

    @functools.partial(
        pl.kernel,
        out_type=[
            jax.ShapeDtypeStruct((NN, DD), jnp.float32),
            jax.ShapeDtypeStruct((NN, DD), jnp.float32),
        ],
        mesh=mesh,
        scratch_types=[
            pltpu.VMEM((NCH, CH), jnp.int32),
            pltpu.VMEM((NCH, CH), jnp.int32),
            [pltpu.VMEM((CH, DD), jnp.float32) for _ in range(NBUF)],
            [pltpu.SemaphoreType.DMA for _ in range(NBUF)],
            [pltpu.SemaphoreType.DMA for _ in range(NBUF)],
            pltpu.VMEM_SHARED((NN, DD), jnp.float32),
            pltpu.SemaphoreType.DMA,
        ],
        compiler_params=pltpu.CompilerParams(use_tc_tiling_on_sc=False),
    )
    def spmm(table_hbm, src_hbm, dst_hbm, zer_hbm, out0, out1,
             src_v, dst_v, bufs, semg, sems, acc, semi):
        cid = lax.axis_index("c")
        sid = lax.axis_index("s")
        wid = cid * NS + sid
        base = pl.multiple_of(sid * SLAB, 8)

        cpi0 = pltpu.async_copy(src_hbm.at[wid], src_v, semi)
        cpi1 = pltpu.async_copy(dst_hbm.at[wid], dst_v, semi)

        zb = bufs[NBUF - 1]
        nz = SLAB // CH
        rem = SLAB - nz * CH
        pltpu.sync_copy(zer_hbm, zb)
        cpi0.wait()
        for b in range(NBUF - 1):
            pltpu.async_copy(table_hbm.at[src_v.at[b]], bufs[b], semg[b])

        for k in range(nz):
            off = pl.multiple_of(base + k * CH, 8)
            pltpu.async_copy(zb, acc.at[pl.ds(off, CH)], sems[k % NBUF])
        offr = pl.multiple_of(base + nz * CH, 8)
        pltpu.async_copy(zb.at[pl.ds(0, rem)], acc.at[pl.ds(offr, rem)],
                         sems[nz % NBUF])

        @pl.when(sid == NS - 1)
        def _():
            off = pl.multiple_of(base + SLAB, 8)
            pltpu.sync_copy(zb.at[pl.ds(0, SLAB_LAST - SLAB)],
                            acc.at[pl.ds(off, SLAB_LAST - SLAB)])

        for k in range(nz):
            off = pl.multiple_of(base + k * CH, 8)
            pltpu.make_async_copy(zb, acc.at[pl.ds(off, CH)],
                                  sems[k % NBUF]).wait()
        pltpu.make_async_copy(zb.at[pl.ds(0, rem)],
                              acc.at[pl.ds(offr, rem)],
                              sems[nz % NBUF]).wait()
        cpi1.wait()
        plsc.subcore_barrier()

        pltpu.async_copy(table_hbm.at[src_v.at[NBUF - 1]],
                         bufs[NBUF - 1], semg[NBUF - 1])

        def body(g, carry):
            c0 = g * NBUF
            for b in range(NBUF):
                c = c0 + b
                pltpu.make_async_copy(table_hbm.at[src_v.at[c]],
                                      bufs[b], semg[b]).wait()
                pltpu.async_copy(bufs[b], acc.at[dst_v.at[c]], sems[b],
                                 add=True)
            for b in range(NBUF):
                nc = c0 + b + NBUF

                @pl.when(nc < NCH)
                def _():
                    pltpu.make_async_copy(bufs[b], acc.at[dst_v.at[c0 + b]],
                                          sems[b]).wait()
                    pltpu.async_copy(table_hbm.at[src_v.at[nc]],
                                     bufs[b], semg[b])
            return carry

        lax.fori_loop(0, NCH // NBUF, body, 0)
        for c in range(NFULL, NCH):
            b = c % NBUF
            pltpu.make_async_copy(table_hbm.at[src_v.at[c]],
                                  bufs[b], semg[b]).wait()
            pltpu.async_copy(bufs[b], acc.at[dst_v.at[c]], sems[b], add=True)
        for c in range(NCH - NBUF, NCH):
            b = c % NBUF
            pltpu.make_async_copy(bufs[b], acc.at[dst_v.at[c]],
                                  sems[b]).wait()
        plsc.subcore_barrier()

        def writeback(out):
            @pl.when(sid < NS - 1)
            def _():
                pltpu.sync_copy(acc.at[pl.ds(base, SLAB)],
                                out.at[pl.ds(base, SLAB)])

            @pl.when(sid == NS - 1)
            def _():
                pltpu.sync_copy(acc.at[pl.ds(base, SLAB_LAST)],
                                out.at[pl.ds(base, SLAB_LAST)])

        @pl.when(cid == 0)
        def _():
            writeback(out0)

        @pl.when(cid == 1)
        def _():
            writeback(out1)

    return spmm(table, src3, dst3, zer)


def _combine_mm(p0, p1, W, b, relu):
    def body(p0_ref, p1_ref, w_ref, b_ref, o_ref):
        s = jnp.dot(p0_ref[...] + p1_ref[...], w_ref[...],
                    preferred_element_type=jnp.float32) + b_ref[...]
        if relu:
            s = jnp.maximum(s, 0.0)
        o_ref[...] = s

    return pl.pallas_call(
        body,
        grid=(NN // MROWS,),
        in_specs=[
            pl.BlockSpec((MROWS, DD), lambda i: (i, 0)),
            pl.BlockSpec((MROWS, DD), lambda i: (i, 0)),
            pl.BlockSpec((DD, DD), lambda i: (0, 0)),
            pl.BlockSpec((1, DD), lambda i: (0, 0)),
        ],
        out_specs=pl.BlockSpec((MROWS, DD), lambda i: (i, 0)),
        out_shape=jax.ShapeDtypeStruct((NN, DD), jnp.float32),
    )(p0, p1, W, b)


def kernel(x, edge_index, W1, b1, W2, b2):
    src3 = edge_index[0].reshape(NW, NCH, CH)
    dst3 = edge_index[1].reshape(NW, NCH, CH)
    zer = jnp.zeros((CH, DD), jnp.float32)
    p0, p1 = _sc_spmm(x, src3, dst3, zer)
    h = _combine_mm(p0, p1, W1, b1.reshape(1, DD), relu=True)
    q0, q1 = _sc_spmm(h, src3, dst3, zer)
    return _combine_mm(q0, q1, W2, b2.reshape(1, DD), relu=False)


# speedup vs baseline: 1.0252x; 1.0194x over previous
import functools

import jax
import jax.numpy as jnp
from jax import lax
from jax.experimental import pallas as pl
from jax.experimental.pallas import tpu as pltpu
from jax.experimental.pallas import tpu_sc as plsc

NN = 10000
EE = 320000
DD = 128
NC = 2
NS = 16
NW = NC * NS
EPW = EE // NW
CH = 40
NCH = EPW // CH
SLAB = 624
SLAB_LAST = NN - SLAB * (NS - 1)
NBUF = 6
NFULL = (NCH // NBUF) * NBUF
MROWS = 1000


def _sc_spmm(table, src3, dst3, zer):
    mesh = plsc.VectorSubcoreMesh(core_axis_name="c", subcore_axis_name="s")

    @functools.partial(
        pl.kernel,
        out_type=[
            jax.ShapeDtypeStruct((NN, DD), jnp.float32),
            jax.ShapeDtypeStruct((NN, DD), jnp.float32),
        ],
        mesh=mesh,
        scratch_types=[
            pltpu.VMEM((NCH, CH), jnp.int32),
            pltpu.VMEM((NCH, CH), jnp.int32),
            [pltpu.VMEM((CH, DD), jnp.float32) for _ in range(NBUF)],
            [pltpu.SemaphoreType.DMA for _ in range(NBUF)],
            [pltpu.SemaphoreType.DMA for _ in range(NBUF)],
            pltpu.VMEM_SHARED((NN, DD), jnp.float32),
            pltpu.SemaphoreType.DMA,
        ],
        compiler_params=pltpu.CompilerParams(use_tc_tiling_on_sc=False),
    )
    def spmm(table_hbm, src_hbm, dst_hbm, zer_hbm, out0, out1,
             src_v, dst_v, bufs, semg, sems, acc, semi):
        cid = lax.axis_index("c")
        sid = lax.axis_index("s")
        wid = cid * NS + sid
        base = pl.multiple_of(sid * SLAB, 8)

        cpi0 = pltpu.async_copy(src_hbm.at[wid], src_v, semi)
        cpi1 = pltpu.async_copy(dst_hbm.at[wid], dst_v, semi)

        zb = bufs[NBUF - 1]
        nz = SLAB // CH
        rem = SLAB - nz * CH
        pltpu.sync_copy(zer_hbm, zb)
        cpi0.wait()
        for b in range(NBUF - 1):
            pltpu.async_copy(table_hbm.at[src_v.at[b]], bufs[b], semg[b])

        for k in range(nz):
            off = pl.multiple_of(base + k * CH, 8)
            pltpu.async_copy(zb, acc.at[pl.ds(off, CH)], sems[k % NBUF])
        offr = pl.multiple_of(base + nz * CH, 8)
        pltpu.async_copy(zb.at[pl.ds(0, rem)], acc.at[pl.ds(offr, rem)],
                         sems[nz % NBUF])

        @pl.when(sid == NS - 1)
        def _():
            off = pl.multiple_of(base + SLAB, 8)
            pltpu.sync_copy(zb.at[pl.ds(0, SLAB_LAST - SLAB)],
                            acc.at[pl.ds(off, SLAB_LAST - SLAB)])

        for k in range(nz):
            off = pl.multiple_of(base + k * CH, 8)
            pltpu.make_async_copy(zb, acc.at[pl.ds(off, CH)],
                                  sems[k % NBUF]).wait()
        pltpu.make_async_copy(zb.at[pl.ds(0, rem)],
                              acc.at[pl.ds(offr, rem)],
                              sems[nz % NBUF]).wait()
        cpi1.wait()
        plsc.subcore_barrier()

        pltpu.async_copy(table_hbm.at[src_v.at[NBUF - 1]],
                         bufs[NBUF - 1], semg[NBUF - 1])

        def body(g, carry):
            c0 = g * NBUF
            for b in range(NBUF):
                c = c0 + b
                pltpu.make_async_copy(table_hbm.at[src_v.at[c]],
                                      bufs[b], semg[b]).wait()
                pltpu.async_copy(bufs[b], acc.at[dst_v.at[c]], sems[b],
                                 add=True)
            for b in range(NBUF):
                nc = c0 + b + NBUF

                @pl.when(nc < NCH)
                def _():
                    pltpu.make_async_copy(bufs[b], acc.at[dst_v.at[c0 + b]],
                                          sems[b]).wait()
                    pltpu.async_copy(table_hbm.at[src_v.at[nc]],
                                     bufs[b], semg[b])
            return carry

        lax.fori_loop(0, NCH // NBUF, body, 0)
        for c in range(NFULL, NCH):
            b = c % NBUF
            pltpu.make_async_copy(table_hbm.at[src_v.at[c]],
                                  bufs[b], semg[b]).wait()
            pltpu.async_copy(bufs[b], acc.at[dst_v.at[c]], sems[b], add=True)
        for c in range(NCH - NBUF, NCH):
            b = c % NBUF
            pltpu.make_async_copy(bufs[b], acc.at[dst_v.at[c]],
                                  sems[b]).wait()
        plsc.subcore_barrier()

        def writeback(out):
            @pl.when(sid < NS - 1)
            def _():
                pltpu.sync_copy(acc.at[pl.ds(base, SLAB)],
                                out.at[pl.ds(base, SLAB)])

            @pl.when(sid == NS - 1)
            def _():
                pltpu.sync_copy(acc.at[pl.ds(base, SLAB_LAST)],
                                out.at[pl.ds(base, SLAB_LAST)])

        @pl.when(cid == 0)
        def _():
            writeback(out0)

        @pl.when(cid == 1)
        def _():
            writeback(out1)

    return spmm(table, src3, dst3, zer)


def _combine_mm(p0, p1, W, b, relu):
    def body(p0_ref, p1_ref, w_ref, b_ref, o_ref):
        s = jnp.dot(p0_ref[...] + p1_ref[...], w_ref[...],
                    preferred_element_type=jnp.float32) + b_ref[...]
        if relu:
            s = jnp.maximum(s, 0.0)
        o_ref[...] = s

    return pl.pallas_call(
        body,
        grid=(NN // MROWS,),
        in_specs=[
            pl.BlockSpec((MROWS, DD), lambda i: (i, 0)),
            pl.BlockSpec((MROWS, DD), lambda i: (i, 0)),
            pl.BlockSpec((DD, DD), lambda i: (0, 0)),
            pl.BlockSpec((1, DD), lambda i: (0, 0)),
        ],
        out_specs=pl.BlockSpec((MROWS, DD), lambda i: (i, 0)),
        out_shape=jax.ShapeDtypeStruct((NN, DD), jnp.float32),
    )(p0, p1, W, b)


def kernel(x, edge_index, W1, b1, W2, b2):
    src3 = edge_index[0].reshape(NW, NCH, CH)
    dst3 = edge_index[1].reshape(NW, NCH, CH)
    zer = jnp.zeros((CH, DD), jnp.float32)
    p0, p1 = _sc_spmm(x, src3, dst3, zer)
    h = _combine_mm(p0, p1, W1, b1.reshape(1, DD), relu=True)
    q0, q1 = _sc_spmm(h, src3, dst3, zer)
    return _combine_mm(q0, q1, W2, b2.reshape(1, DD), relu=False)


# speedup vs baseline: 1.0466x; 1.0209x over previous
import functools

import jax
import jax.numpy as jnp
from jax import lax
from jax.experimental import pallas as pl
from jax.experimental.pallas import tpu as pltpu
from jax.experimental.pallas import tpu_sc as plsc

NN = 10000
EE = 320000
DD = 128
NC = 2
NS = 16
NW = NC * NS
EPW = EE // NW
CH = 40
NCH = EPW // CH
SLAB = 624
SLAB_LAST = NN - SLAB * (NS - 1)
NBUF = 6
NFULL = (NCH // NBUF) * NBUF
MROWS = 2000


def _sc_spmm(table, src3, dst3, zer):
    mesh = plsc.VectorSubcoreMesh(core_axis_name="c", subcore_axis_name="s")

    @functools.partial(
        pl.kernel,
        out_type=[
            jax.ShapeDtypeStruct((NN, DD), jnp.float32),
            jax.ShapeDtypeStruct((NN, DD), jnp.float32),
        ],
        mesh=mesh,
        scratch_types=[
            pltpu.VMEM((NCH, CH), jnp.int32),
            pltpu.VMEM((NCH, CH), jnp.int32),
            [pltpu.VMEM((CH, DD), jnp.float32) for _ in range(NBUF)],
            [pltpu.SemaphoreType.DMA for _ in range(NBUF)],
            [pltpu.SemaphoreType.DMA for _ in range(NBUF)],
            pltpu.VMEM_SHARED((NN, DD), jnp.float32),
            pltpu.SemaphoreType.DMA,
        ],
        compiler_params=pltpu.CompilerParams(use_tc_tiling_on_sc=False),
    )
    def spmm(table_hbm, src_hbm, dst_hbm, zer_hbm, out0, out1,
             src_v, dst_v, bufs, semg, sems, acc, semi):
        cid = lax.axis_index("c")
        sid = lax.axis_index("s")
        wid = cid * NS + sid
        base = pl.multiple_of(sid * SLAB, 8)

        cpi0 = pltpu.async_copy(src_hbm.at[wid], src_v, semi)
        cpi1 = pltpu.async_copy(dst_hbm.at[wid], dst_v, semi)

        zb = bufs[NBUF - 1]
        nz = SLAB // CH
        rem = SLAB - nz * CH
        pltpu.sync_copy(zer_hbm, zb)
        cpi0.wait()
        for b in range(NBUF - 1):
            pltpu.async_copy(table_hbm.at[src_v.at[b]], bufs[b], semg[b])

        for k in range(nz):
            off = pl.multiple_of(base + k * CH, 8)
            pltpu.async_copy(zb, acc.at[pl.ds(off, CH)], sems[k % NBUF])
        offr = pl.multiple_of(base + nz * CH, 8)
        pltpu.async_copy(zb.at[pl.ds(0, rem)], acc.at[pl.ds(offr, rem)],
                         sems[nz % NBUF])

        @pl.when(sid == NS - 1)
        def _():
            off = pl.multiple_of(base + SLAB, 8)
            pltpu.sync_copy(zb.at[pl.ds(0, SLAB_LAST - SLAB)],
                            acc.at[pl.ds(off, SLAB_LAST - SLAB)])

        for k in range(nz):
            off = pl.multiple_of(base + k * CH, 8)
            pltpu.make_async_copy(zb, acc.at[pl.ds(off, CH)],
                                  sems[k % NBUF]).wait()
        pltpu.make_async_copy(zb.at[pl.ds(0, rem)],
                              acc.at[pl.ds(offr, rem)],
                              sems[nz % NBUF]).wait()
        cpi1.wait()
        plsc.subcore_barrier()

        pltpu.async_copy(table_hbm.at[src_v.at[NBUF - 1]],
                         bufs[NBUF - 1], semg[NBUF - 1])

        def body(g, carry):
            c0 = g * NBUF
            for b in range(NBUF):
                c = c0 + b
                pltpu.make_async_copy(table_hbm.at[src_v.at[c]],
                                      bufs[b], semg[b]).wait()
                pltpu.async_copy(bufs[b], acc.at[dst_v.at[c]], sems[b],
                                 add=True)
            for b in range(NBUF):
                nc = c0 + b + NBUF

                @pl.when(nc < NCH)
                def _():
                    pltpu.make_async_copy(bufs[b], acc.at[dst_v.at[c0 + b]],
                                          sems[b]).wait()
                    pltpu.async_copy(table_hbm.at[src_v.at[nc]],
                                     bufs[b], semg[b])
            return carry

        lax.fori_loop(0, NCH // NBUF, body, 0)
        for c in range(NFULL, NCH):
            b = c % NBUF
            pltpu.make_async_copy(table_hbm.at[src_v.at[c]],
                                  bufs[b], semg[b]).wait()
            pltpu.async_copy(bufs[b], acc.at[dst_v.at[c]], sems[b], add=True)
        for c in range(NCH - NBUF, NCH):
            b = c % NBUF
            pltpu.make_async_copy(bufs[b], acc.at[dst_v.at[c]],
                                  sems[b]).wait()
        plsc.subcore_barrier()

        def writeback(out):
            @pl.when(sid < NS - 1)
            def _():
                pltpu.sync_copy(acc.at[pl.ds(base, SLAB)],
                                out.at[pl.ds(base, SLAB)])

            @pl.when(sid == NS - 1)
            def _():
                pltpu.sync_copy(acc.at[pl.ds(base, SLAB_LAST)],
                                out.at[pl.ds(base, SLAB_LAST)])

        @pl.when(cid == 0)
        def _():
            writeback(out0)

        @pl.when(cid == 1)
        def _():
            writeback(out1)

    return spmm(table, src3, dst3, zer)


def _combine_mm(p0, p1, W, b, relu):
    def body(p0_ref, p1_ref, w_ref, b_ref, o_ref):
        s = jnp.dot(p0_ref[...] + p1_ref[...], w_ref[...],
                    preferred_element_type=jnp.float32) + b_ref[...]
        if relu:
            s = jnp.maximum(s, 0.0)
        o_ref[...] = s

    return pl.pallas_call(
        body,
        grid=(NN // MROWS,),
        in_specs=[
            pl.BlockSpec((MROWS, DD), lambda i: (i, 0)),
            pl.BlockSpec((MROWS, DD), lambda i: (i, 0)),
            pl.BlockSpec((DD, DD), lambda i: (0, 0)),
            pl.BlockSpec((1, DD), lambda i: (0, 0)),
        ],
        out_specs=pl.BlockSpec((MROWS, DD), lambda i: (i, 0)),
        out_shape=jax.ShapeDtypeStruct((NN, DD), jnp.float32),
    )(p0, p1, W, b)


def kernel(x, edge_index, W1, b1, W2, b2):
    src3 = edge_index[0].reshape(NW, NCH, CH)
    dst3 = edge_index[1].reshape(NW, NCH, CH)
    zer = jnp.zeros((CH, DD), jnp.float32)
    p0, p1 = _sc_spmm(x, src3, dst3, zer)
    h = _combine_mm(p0, p1, W1, b1.reshape(1, DD), relu=True)
    q0, q1 = _sc_spmm(h, src3, dst3, zer)
    return _combine_mm(q0, q1, W2, b2.reshape(1, DD), relu=False)


# speedup vs baseline: 1.0543x; 1.0073x over previous
import functools

import jax
import jax.numpy as jnp
from jax import lax
from jax.experimental import pallas as pl
from jax.experimental.pallas import tpu as pltpu
from jax.experimental.pallas import tpu_sc as plsc

NN = 10000
EE = 320000
DD = 128
NC = 2
NS = 16
NW = NC * NS
EPW = EE // NW
CH = 40
NCH = EPW // CH
SLAB = 624
SLAB_LAST = NN - SLAB * (NS - 1)
NBUF = 6
NFULL = (NCH // NBUF) * NBUF
MROWS = 10000


def _sc_spmm(table, src3, dst3, zer):
    mesh = plsc.VectorSubcoreMesh(core_axis_name="c", subcore_axis_name="s")

    @functools.partial(
        pl.kernel,
        out_type=[
            jax.ShapeDtypeStruct((NN, DD), jnp.float32),
            jax.ShapeDtypeStruct((NN, DD), jnp.float32),
        ],
        mesh=mesh,
        scratch_types=[
            pltpu.VMEM((NCH, CH), jnp.int32),
            pltpu.VMEM((NCH, CH), jnp.int32),
            [pltpu.VMEM((CH, DD), jnp.float32) for _ in range(NBUF)],
            [pltpu.SemaphoreType.DMA for _ in range(NBUF)],
            [pltpu.SemaphoreType.DMA for _ in range(NBUF)],
            pltpu.VMEM_SHARED((NN, DD), jnp.float32),
            pltpu.SemaphoreType.DMA,
        ],
        compiler_params=pltpu.CompilerParams(use_tc_tiling_on_sc=False),
    )
    def spmm(table_hbm, src_hbm, dst_hbm, zer_hbm, out0, out1,
             src_v, dst_v, bufs, semg, sems, acc, semi):
        cid = lax.axis_index("c")
        sid = lax.axis_index("s")
        wid = cid * NS + sid
        base = pl.multiple_of(sid * SLAB, 8)

        cpi0 = pltpu.async_copy(src_hbm.at[wid], src_v, semi)
        cpi1 = pltpu.async_copy(dst_hbm.at[wid], dst_v, semi)

        zb = bufs[NBUF - 1]
        nz = SLAB // CH
        rem = SLAB - nz * CH
        pltpu.sync_copy(zer_hbm, zb)
        cpi0.wait()
        for b in range(NBUF - 1):
            pltpu.async_copy(table_hbm.at[src_v.at[b]], bufs[b], semg[b])

        for k in range(nz):
            off = pl.multiple_of(base + k * CH, 8)
            pltpu.async_copy(zb, acc.at[pl.ds(off, CH)], sems[k % NBUF])
        offr = pl.multiple_of(base + nz * CH, 8)
        pltpu.async_copy(zb.at[pl.ds(0, rem)], acc.at[pl.ds(offr, rem)],
                         sems[nz % NBUF])

        @pl.when(sid == NS - 1)
        def _():
            off = pl.multiple_of(base + SLAB, 8)
            pltpu.sync_copy(zb.at[pl.ds(0, SLAB_LAST - SLAB)],
                            acc.at[pl.ds(off, SLAB_LAST - SLAB)])

        for k in range(nz):
            off = pl.multiple_of(base + k * CH, 8)
            pltpu.make_async_copy(zb, acc.at[pl.ds(off, CH)],
                                  sems[k % NBUF]).wait()
        pltpu.make_async_copy(zb.at[pl.ds(0, rem)],
                              acc.at[pl.ds(offr, rem)],
                              sems[nz % NBUF]).wait()
        cpi1.wait()
        plsc.subcore_barrier()

        pltpu.async_copy(table_hbm.at[src_v.at[NBUF - 1]],
                         bufs[NBUF - 1], semg[NBUF - 1])

        def body(g, carry):
            c0 = g * NBUF
            for b in range(NBUF):
                c = c0 + b
                pltpu.make_async_copy(table_hbm.at[src_v.at[c]],
                                      bufs[b], semg[b]).wait()
                pltpu.async_copy(bufs[b], acc.at[dst_v.at[c]], sems[b],
                                 add=True)
            for b in range(NBUF):
                nc = c0 + b + NBUF

                @pl.when(nc < NCH)
                def _():
                    pltpu.make_async_copy(bufs[b], acc.at[dst_v.at[c0 + b]],
                                          sems[b]).wait()
                    pltpu.async_copy(table_hbm.at[src_v.at[nc]],
                                     bufs[b], semg[b])
            return carry

        lax.fori_loop(0, NCH // NBUF, body, 0)
        for c in range(NFULL, NCH):
            b = c % NBUF
            pltpu.make_async_copy(table_hbm.at[src_v.at[c]],
                                  bufs[b], semg[b]).wait()
            pltpu.async_copy(bufs[b], acc.at[dst_v.at[c]], sems[b], add=True)
        for c in range(NCH - NBUF, NCH):
            b = c % NBUF
            pltpu.make_async_copy(bufs[b], acc.at[dst_v.at[c]],
                                  sems[b]).wait()
        plsc.subcore_barrier()

        def writeback(out):
            @pl.when(sid < NS - 1)
            def _():
                pltpu.sync_copy(acc.at[pl.ds(base, SLAB)],
                                out.at[pl.ds(base, SLAB)])

            @pl.when(sid == NS - 1)
            def _():
                pltpu.sync_copy(acc.at[pl.ds(base, SLAB_LAST)],
                                out.at[pl.ds(base, SLAB_LAST)])

        @pl.when(cid == 0)
        def _():
            writeback(out0)

        @pl.when(cid == 1)
        def _():
            writeback(out1)

    return spmm(table, src3, dst3, zer)


def _combine_mm(p0, p1, W, b, relu):
    def body(p0_ref, p1_ref, w_ref, b_ref, o_ref):
        s = jnp.dot(p0_ref[...] + p1_ref[...], w_ref[...],
                    preferred_element_type=jnp.float32) + b_ref[...]
        if relu:
            s = jnp.maximum(s, 0.0)
        o_ref[...] = s

    return pl.pallas_call(
        body,
        grid=(NN // MROWS,),
        in_specs=[
            pl.BlockSpec((MROWS, DD), lambda i: (i, 0)),
            pl.BlockSpec((MROWS, DD), lambda i: (i, 0)),
            pl.BlockSpec((DD, DD), lambda i: (0, 0)),
            pl.BlockSpec((1, DD), lambda i: (0, 0)),
        ],
        out_specs=pl.BlockSpec((MROWS, DD), lambda i: (i, 0)),
        out_shape=jax.ShapeDtypeStruct((NN, DD), jnp.float32),
    )(p0, p1, W, b)


def kernel(x, edge_index, W1, b1, W2, b2):
    src3 = edge_index[0].reshape(NW, NCH, CH)
    dst3 = edge_index[1].reshape(NW, NCH, CH)
    zer = jnp.zeros((CH, DD), jnp.float32)
    p0, p1 = _sc_spmm(x, src3, dst3, zer)
    h = _combine_mm(p0, p1, W1, b1.reshape(1, DD), relu=True)
    q0, q1 = _sc_spmm(h, src3, dst3, zer)
    return _combine_mm(q0, q1, W2, b2.reshape(1, DD), relu=False)
